# baseline (speedup 1.0000x reference)
import functools

import jax
import jax.numpy as jnp
from jax import lax
from jax.experimental import pallas as pl
from jax.experimental.pallas import tpu as pltpu
from jax.experimental.pallas import tpu_sc as plsc

N = 10000
E = 320000
G = 256
D = 128

NPAD = 10240
K = 128
CHUNKS = 80
EPAD = 32 * CHUNKS * K
NB = 12288
GPAD = 384

f32 = jnp.float32
i32 = jnp.int32

_MESH = plsc.VectorSubcoreMesh(core_axis_name="c", subcore_axis_name="s")


@functools.partial(
    pl.kernel,
    out_type=(jax.ShapeDtypeStruct((2, NPAD, D), f32),
              jax.ShapeDtypeStruct((2, GPAD, D), f32)),
    mesh=_MESH,
    scratch_types=[
        pltpu.VMEM((CHUNKS, K), i32),
        pltpu.VMEM((3, 128), i32),
        pltpu.VMEM((128, D), f32),
        pltpu.VMEM_SHARED((NPAD, D), f32),
        pltpu.VMEM_SHARED((GPAD, D), f32),
    ],
)
def _deg_kernel(dst_hbm, batch_hbm, ones_hbm, zeros_hbm, deg_out, cnt_out,
                dstv, batchv, onesv, degsh, cntsh):
    c = lax.axis_index("c")
    s = lax.axis_index("s")
    wid = s * 2 + c
    pltpu.sync_copy(dst_hbm.at[wid], dstv)
    pltpu.sync_copy(batch_hbm.at[wid], batchv)
    pltpu.sync_copy(ones_hbm, onesv)
    pltpu.sync_copy(zeros_hbm, degsh.at[pl.ds(s * 640, 640)])
    pltpu.sync_copy(zeros_hbm.at[pl.ds(0, 24)], cntsh.at[pl.ds(s * 24, 24)])
    plsc.subcore_barrier()

    @pl.loop(0, CHUNKS)
    def _(j):
        pltpu.sync_copy(onesv, degsh.at[dstv.at[j]], add=True)

    @pl.loop(0, 3)
    def _(j):
        pltpu.sync_copy(onesv, cntsh.at[batchv.at[j]], add=True)

    plsc.subcore_barrier()
    pltpu.sync_copy(degsh.at[pl.ds(s * 640, 640)],
                    deg_out.at[c, pl.ds(s * 640, 640)])
    pltpu.sync_copy(cntsh.at[pl.ds(s * 24, 24)],
                    cnt_out.at[c, pl.ds(s * 24, 24)])


NC0 = 104
NC1 = 56


@functools.partial(
    pl.kernel,
    out_type=jax.ShapeDtypeStruct((2, NPAD, D), f32),
    mesh=_MESH,
    scratch_types=[
        pltpu.VMEM((16, K), i32),
        pltpu.VMEM((8, K), i32),
        pltpu.VMEM((K, D), f32),
        pltpu.VMEM((K, D), f32),
        pltpu.VMEM_SHARED((NPAD, D), f32),
        pltpu.SemaphoreType.DMA,
        pltpu.SemaphoreType.DMA,
        pltpu.SemaphoreType.DMA,
        pltpu.SemaphoreType.DMA,
    ],
)
def _agg_kernel(h_hbm, src0_hbm, dst0_hbm, src1_hbm, dst1_hbm, zeros_hbm,
                out_hbm, srcr, dstr, buf_a, buf_b, aggsh, sga, sgb, ssa, ssb):
    c = lax.axis_index("c")
    s = lax.axis_index("s")
    pltpu.sync_copy(zeros_hbm, aggsh.at[pl.ds(s * 640, 640)])
    plsc.subcore_barrier()

    H = K // 2

    def gather(j, buf, sem):
        r = j & 15
        pltpu.async_copy(h_hbm.at[srcr.at[r, pl.ds(0, H)]],
                         buf.at[pl.ds(0, H)], sem)
        pltpu.async_copy(h_hbm.at[srcr.at[r, pl.ds(H, H)]],
                         buf.at[pl.ds(H, H)], sem)

    def wait_gather(buf, sem):
        pltpu.make_async_copy(h_hbm.at[srcr.at[0, pl.ds(0, H)]],
                              buf.at[pl.ds(0, H)], sem).wait()
        pltpu.make_async_copy(h_hbm.at[srcr.at[0, pl.ds(0, H)]],
                              buf.at[pl.ds(0, H)], sem).wait()

    def wait_scatter(buf, sem):
        pltpu.make_async_copy(buf, aggsh.at[dstr.at[0]], sem).wait()

    def pipeline(nch, src_hbm, dst_hbm):
        ngroups = nch // 8
        pltpu.sync_copy(src_hbm.at[s, pl.ds(0, 16)], srcr)
        pltpu.sync_copy(dst_hbm.at[s, pl.ds(0, 8)], dstr)
        gather(0, buf_a, sga)

        @pl.loop(0, ngroups)
        def _(g):
            j0 = g * 8
            for b in range(0, 8, 2):
                j = j0 + b
                gather(j + 1, buf_b, sgb)
                wait_gather(buf_a, sga)
                pltpu.async_copy(buf_a, aggsh.at[dstr.at[b]], ssa, add=True)
                wait_scatter(buf_a, ssa)

                @pl.when(j + 2 < nch)
                def _():
                    gather(j + 2, buf_a, sga)

                wait_gather(buf_b, sgb)
                pltpu.async_copy(buf_b, aggsh.at[dstr.at[b + 1]], ssb,
                                 add=True)
                wait_scatter(buf_b, ssb)

            @pl.when(g < ngroups - 1)
            def _():
                pltpu.sync_copy(dst_hbm.at[s, pl.ds(j0 + 8, 8)], dstr)

            @pl.when(g < ngroups - 2)
            def _():
                half = (g & 1) * 8
                pltpu.sync_copy(src_hbm.at[s, pl.ds(j0 + 16, 8)],
                                srcr.at[pl.ds(half, 8)])

        plsc.subcore_barrier()
        pltpu.sync_copy(aggsh.at[pl.ds(s * 640, 640)],
                        out_hbm.at[c, pl.ds(s * 640, 640)])

    @pl.when(c == 0)
    def _():
        pipeline(NC0, src0_hbm, dst0_hbm)

    @pl.when(c == 1)
    def _():
        pipeline(NC1, src1_hbm, dst1_hbm)


@functools.partial(
    pl.kernel,
    out_type=(jax.ShapeDtypeStruct((2, GPAD, D), f32),
              jax.ShapeDtypeStruct((G, D), f32)),
    mesh=_MESH,
    scratch_types=[
        pltpu.VMEM((320, D), f32),
        pltpu.VMEM((4, 80), i32),
        pltpu.VMEM((128,), i32),
        pltpu.VMEM((128, D), f32),
        pltpu.VMEM_SHARED((GPAD, D), f32),
        pltpu.SemaphoreType.DMA,
    ],
)
def _pool_kernel(h_hbm, batch_hbm, root_hbm, zeros_hbm, pool_out, root_out,
                 hv, bidxv, ridxv, rootbuf, poolsh, sem):
    c = lax.axis_index("c")
    s = lax.axis_index("s")
    wid = s * 2 + c
    pltpu.sync_copy(zeros_hbm.at[pl.ds(0, 24)], poolsh.at[pl.ds(s * 24, 24)])
    pltpu.sync_copy(h_hbm.at[pl.ds(wid * 320, 320)], hv)
    pltpu.sync_copy(batch_hbm.at[wid], bidxv)
    plsc.subcore_barrier()

    @pl.loop(0, 4)
    def _(j):
        pltpu.sync_copy(hv.at[pl.ds(j * 80, 80)], poolsh.at[bidxv.at[j]],
                        add=True)

    plsc.subcore_barrier()
    pltpu.sync_copy(poolsh.at[pl.ds(s * 24, 24)],
                    pool_out.at[c, pl.ds(s * 24, 24)])

    @pl.when(wid < 2)
    def _():
        pltpu.sync_copy(root_hbm.at[wid], ridxv)
        pltpu.async_copy(h_hbm.at[ridxv], rootbuf, sem).wait()
        pltpu.sync_copy(rootbuf, root_out.at[pl.ds(wid * 128, 128)])


BLK = 512


def _layer_tc(agg2, deg2, h, Wl, Wr, b):
    def body(a0, a1, d0, d1, h_ref, wl, wr, bb, out):
        d0a = d0[...][0]
        d1a = d1[...][0]
        deg = d0a[:, 0:1] + d1a[:, 0:1]
        dinv = 1.0 / jnp.maximum(deg, 1.0)
        t = (a0[...][0] + a1[...][0]) * dinv
        acc = jnp.dot(t, wl[...], preferred_element_type=f32)
        acc += jnp.dot(h_ref[...], wr[...], preferred_element_type=f32)
        out[...] = jnp.maximum(acc + bb[...], 0.0)

    return pl.pallas_call(
        body,
        grid=(NPAD // BLK,),
        in_specs=[
            pl.BlockSpec((1, BLK, D), lambda i: (0, i, 0)),
            pl.BlockSpec((1, BLK, D), lambda i: (1, i, 0)),
            pl.BlockSpec((1, BLK, D), lambda i: (0, i, 0)),
            pl.BlockSpec((1, BLK, D), lambda i: (1, i, 0)),
            pl.BlockSpec((BLK, D), lambda i: (i, 0)),
            pl.BlockSpec((D, D), lambda i: (0, 0)),
            pl.BlockSpec((D, D), lambda i: (0, 0)),
            pl.BlockSpec((1, D), lambda i: (0, 0)),
        ],
        out_specs=pl.BlockSpec((BLK, D), lambda i: (i, 0)),
        out_shape=jax.ShapeDtypeStruct((NPAD, D), f32),
    )(agg2, agg2, deg2, deg2, h, Wl, Wr, b)


def _final_tc(root, pool2, cnt2, Wr1, Wr2, blin):
    def body(r, p0, p1, c0, c1, w1, w2, bb, out):
        p0a = p0[...][0]
        p1a = p1[...][0]
        cnt = c0[...][0][:, 0:1] + c1[...][0][:, 0:1]
        pooled = (p0a + p1a) / jnp.maximum(cnt, 1.0)
        acc = jnp.dot(r[...], w1[...], preferred_element_type=f32)
        acc += jnp.dot(pooled, w2[...], preferred_element_type=f32)
        out[...] = acc + bb[...]

    return pl.pallas_call(
        body,
        grid=(1,),
        in_specs=[
            pl.BlockSpec((G, D), lambda i: (0, 0)),
            pl.BlockSpec((1, G, D), lambda i: (0, 0, 0)),
            pl.BlockSpec((1, G, D), lambda i: (1, 0, 0)),
            pl.BlockSpec((1, G, D), lambda i: (0, 0, 0)),
            pl.BlockSpec((1, G, D), lambda i: (1, 0, 0)),
            pl.BlockSpec((D, D), lambda i: (0, 0)),
            pl.BlockSpec((D, D), lambda i: (0, 0)),
            pl.BlockSpec((1, D), lambda i: (0, 0)),
        ],
        out_specs=pl.BlockSpec((G, D), lambda i: (0, 0)),
        out_shape=jax.ShapeDtypeStruct((G, D), f32),
    )(root, pool2, pool2, cnt2, cnt2, Wr1, Wr2, blin)


def kernel(x, edge_index, batch, root_n_id,
           W1l, W1r, b1, W2l, W2r, b2, W3l, W3r, b3, Wlin, blin):
    src = edge_index[0]
    dst = edge_index[1]
    src_p = jnp.concatenate(
        [src, jnp.zeros((EPAD - E,), i32)]).reshape(32, CHUNKS, K)
    dst_p = jnp.concatenate(
        [dst, jnp.full((EPAD - E,), N, i32)]).reshape(32, CHUNKS, K)
    batch_c = jnp.concatenate(
        [batch, jnp.full((NB - N,), G, i32)]).reshape(32, 3, 128)
    batch_s = jnp.concatenate(
        [batch, jnp.full((NPAD - N,), G, i32)]).reshape(32, 4, 80)
    root_p = root_n_id.reshape(2, 128)
    zeros128 = jnp.zeros((640, D), f32)
    ones128 = jnp.ones((128, D), f32)
    x_p = jnp.pad(x, ((0, NPAD - N), (0, 0)))

    deg2, cnt2 = _deg_kernel(dst_p, batch_c, ones128, zeros128)

    srcc = src_p.reshape(EPAD // K, K)
    dstc = dst_p.reshape(EPAD // K, K)
    src0 = srcc[:16 * NC0].reshape(16, NC0, K)
    dst0 = dstc[:16 * NC0].reshape(16, NC0, K)
    src1 = srcc[16 * NC0:].reshape(16, NC1, K)
    dst1 = dstc[16 * NC0:].reshape(16, NC1, K)

    h = x_p
    for Wl, Wr, b in ((W1l, W1r, b1), (W2l, W2r, b2), (W3l, W3r, b3)):
        agg2 = _agg_kernel(h, src0, dst0, src1, dst1, zeros128)
        h = _layer_tc(agg2, deg2, h, Wl, Wr, b.reshape(1, D))

    pool2, root = _pool_kernel(h, batch_s, root_p, zeros128)
    return _final_tc(root, pool2, cnt2, Wlin[:D], Wlin[D:], blin.reshape(1, D))


# speedup vs baseline: 1.0235x; 1.0235x over previous
import functools

import jax
import jax.numpy as jnp
from jax import lax
from jax.experimental import pallas as pl
from jax.experimental.pallas import tpu as pltpu
from jax.experimental.pallas import tpu_sc as plsc

N = 10000
E = 320000
G = 256
D = 128

NPAD = 10240
K = 128
CHUNKS = 80
EPAD = 32 * CHUNKS * K
NB = 12288
GPAD = 384

f32 = jnp.float32
i32 = jnp.int32

_MESH = plsc.VectorSubcoreMesh(core_axis_name="c", subcore_axis_name="s")


@functools.partial(
    pl.kernel,
    out_type=(jax.ShapeDtypeStruct((2, NPAD, D), f32),
              jax.ShapeDtypeStruct((2, GPAD, D), f32)),
    mesh=_MESH,
    scratch_types=[
        pltpu.VMEM((CHUNKS, K), i32),
        pltpu.VMEM((3, 128), i32),
        pltpu.VMEM((128, D), f32),
        pltpu.VMEM_SHARED((NPAD, D), f32),
        pltpu.VMEM_SHARED((GPAD, D), f32),
    ],
)
def _deg_kernel(dst_hbm, batch_hbm, ones_hbm, zeros_hbm, deg_out, cnt_out,
                dstv, batchv, onesv, degsh, cntsh):
    c = lax.axis_index("c")
    s = lax.axis_index("s")
    wid = s * 2 + c
    pltpu.sync_copy(dst_hbm.at[wid], dstv)
    pltpu.sync_copy(batch_hbm.at[wid], batchv)
    pltpu.sync_copy(ones_hbm, onesv)
    pltpu.sync_copy(zeros_hbm, degsh.at[pl.ds(s * 640, 640)])
    pltpu.sync_copy(zeros_hbm.at[pl.ds(0, 24)], cntsh.at[pl.ds(s * 24, 24)])
    plsc.subcore_barrier()

    @pl.loop(0, CHUNKS)
    def _(j):
        pltpu.sync_copy(onesv, degsh.at[dstv.at[j]], add=True)

    @pl.loop(0, 3)
    def _(j):
        pltpu.sync_copy(onesv, cntsh.at[batchv.at[j]], add=True)

    plsc.subcore_barrier()
    pltpu.sync_copy(degsh.at[pl.ds(s * 640, 640)],
                    deg_out.at[c, pl.ds(s * 640, 640)])
    pltpu.sync_copy(cntsh.at[pl.ds(s * 24, 24)],
                    cnt_out.at[c, pl.ds(s * 24, 24)])


NC0 = 128
NC1 = 32


@functools.partial(
    pl.kernel,
    out_type=jax.ShapeDtypeStruct((2, NPAD, D), f32),
    mesh=_MESH,
    scratch_types=[
        pltpu.VMEM((16, K), i32),
        pltpu.VMEM((8, K), i32),
        pltpu.VMEM((K, D), f32),
        pltpu.VMEM((K, D), f32),
        pltpu.VMEM_SHARED((NPAD, D), f32),
        pltpu.SemaphoreType.DMA,
        pltpu.SemaphoreType.DMA,
        pltpu.SemaphoreType.DMA,
        pltpu.SemaphoreType.DMA,
    ],
)
def _agg_kernel(h_hbm, src0_hbm, dst0_hbm, src1_hbm, dst1_hbm, zeros_hbm,
                out_hbm, srcr, dstr, buf_a, buf_b, aggsh, sga, sgb, ssa, ssb):
    c = lax.axis_index("c")
    s = lax.axis_index("s")
    pltpu.sync_copy(zeros_hbm, aggsh.at[pl.ds(s * 640, 640)])
    plsc.subcore_barrier()

    H = K // 2

    def gather(j, buf, sem):
        r = j & 15
        pltpu.async_copy(h_hbm.at[srcr.at[r, pl.ds(0, H)]],
                         buf.at[pl.ds(0, H)], sem)
        pltpu.async_copy(h_hbm.at[srcr.at[r, pl.ds(H, H)]],
                         buf.at[pl.ds(H, H)], sem)

    def wait_gather(buf, sem):
        pltpu.make_async_copy(h_hbm.at[srcr.at[0, pl.ds(0, H)]],
                              buf.at[pl.ds(0, H)], sem).wait()
        pltpu.make_async_copy(h_hbm.at[srcr.at[0, pl.ds(0, H)]],
                              buf.at[pl.ds(0, H)], sem).wait()

    def wait_scatter(buf, sem):
        pltpu.make_async_copy(buf, aggsh.at[dstr.at[0]], sem).wait()

    def pipeline(nch, src_hbm, dst_hbm):
        ngroups = nch // 8
        pltpu.sync_copy(src_hbm.at[s, pl.ds(0, 16)], srcr)
        pltpu.sync_copy(dst_hbm.at[s, pl.ds(0, 8)], dstr)
        gather(0, buf_a, sga)

        @pl.loop(0, ngroups)
        def _(g):
            j0 = g * 8
            for b in range(0, 8, 2):
                j = j0 + b
                gather(j + 1, buf_b, sgb)
                wait_gather(buf_a, sga)
                pltpu.async_copy(buf_a, aggsh.at[dstr.at[b]], ssa, add=True)
                wait_scatter(buf_a, ssa)

                @pl.when(j + 2 < nch)
                def _():
                    gather(j + 2, buf_a, sga)

                wait_gather(buf_b, sgb)
                pltpu.async_copy(buf_b, aggsh.at[dstr.at[b + 1]], ssb,
                                 add=True)
                wait_scatter(buf_b, ssb)

            @pl.when(g < ngroups - 1)
            def _():
                pltpu.sync_copy(dst_hbm.at[s, pl.ds(j0 + 8, 8)], dstr)

            @pl.when(g < ngroups - 2)
            def _():
                half = (g & 1) * 8
                pltpu.sync_copy(src_hbm.at[s, pl.ds(j0 + 16, 8)],
                                srcr.at[pl.ds(half, 8)])

        plsc.subcore_barrier()
        pltpu.sync_copy(aggsh.at[pl.ds(s * 640, 640)],
                        out_hbm.at[c, pl.ds(s * 640, 640)])

    @pl.when(c == 0)
    def _():
        pipeline(NC0, src0_hbm, dst0_hbm)

    @pl.when(c == 1)
    def _():
        pipeline(NC1, src1_hbm, dst1_hbm)


@functools.partial(
    pl.kernel,
    out_type=(jax.ShapeDtypeStruct((2, GPAD, D), f32),
              jax.ShapeDtypeStruct((G, D), f32)),
    mesh=_MESH,
    scratch_types=[
        pltpu.VMEM((320, D), f32),
        pltpu.VMEM((4, 80), i32),
        pltpu.VMEM((128,), i32),
        pltpu.VMEM((128, D), f32),
        pltpu.VMEM_SHARED((GPAD, D), f32),
        pltpu.SemaphoreType.DMA,
    ],
)
def _pool_kernel(h_hbm, batch_hbm, root_hbm, zeros_hbm, pool_out, root_out,
                 hv, bidxv, ridxv, rootbuf, poolsh, sem):
    c = lax.axis_index("c")
    s = lax.axis_index("s")
    wid = s * 2 + c
    pltpu.sync_copy(zeros_hbm.at[pl.ds(0, 24)], poolsh.at[pl.ds(s * 24, 24)])
    pltpu.sync_copy(h_hbm.at[pl.ds(wid * 320, 320)], hv)
    pltpu.sync_copy(batch_hbm.at[wid], bidxv)
    plsc.subcore_barrier()

    @pl.loop(0, 4)
    def _(j):
        pltpu.sync_copy(hv.at[pl.ds(j * 80, 80)], poolsh.at[bidxv.at[j]],
                        add=True)

    plsc.subcore_barrier()
    pltpu.sync_copy(poolsh.at[pl.ds(s * 24, 24)],
                    pool_out.at[c, pl.ds(s * 24, 24)])

    @pl.when(wid < 2)
    def _():
        pltpu.sync_copy(root_hbm.at[wid], ridxv)
        pltpu.async_copy(h_hbm.at[ridxv], rootbuf, sem).wait()
        pltpu.sync_copy(rootbuf, root_out.at[pl.ds(wid * 128, 128)])


BLK = 512


def _layer_tc(agg2, deg2, h, Wl, Wr, b):
    def body(a0, a1, d0, d1, h_ref, wl, wr, bb, out):
        d0a = d0[...][0]
        d1a = d1[...][0]
        deg = d0a[:, 0:1] + d1a[:, 0:1]
        dinv = 1.0 / jnp.maximum(deg, 1.0)
        t = (a0[...][0] + a1[...][0]) * dinv
        acc = jnp.dot(t, wl[...], preferred_element_type=f32)
        acc += jnp.dot(h_ref[...], wr[...], preferred_element_type=f32)
        out[...] = jnp.maximum(acc + bb[...], 0.0)

    return pl.pallas_call(
        body,
        grid=(NPAD // BLK,),
        in_specs=[
            pl.BlockSpec((1, BLK, D), lambda i: (0, i, 0)),
            pl.BlockSpec((1, BLK, D), lambda i: (1, i, 0)),
            pl.BlockSpec((1, BLK, D), lambda i: (0, i, 0)),
            pl.BlockSpec((1, BLK, D), lambda i: (1, i, 0)),
            pl.BlockSpec((BLK, D), lambda i: (i, 0)),
            pl.BlockSpec((D, D), lambda i: (0, 0)),
            pl.BlockSpec((D, D), lambda i: (0, 0)),
            pl.BlockSpec((1, D), lambda i: (0, 0)),
        ],
        out_specs=pl.BlockSpec((BLK, D), lambda i: (i, 0)),
        out_shape=jax.ShapeDtypeStruct((NPAD, D), f32),
    )(agg2, agg2, deg2, deg2, h, Wl, Wr, b)


def _final_tc(root, pool2, cnt2, Wr1, Wr2, blin):
    def body(r, p0, p1, c0, c1, w1, w2, bb, out):
        p0a = p0[...][0]
        p1a = p1[...][0]
        cnt = c0[...][0][:, 0:1] + c1[...][0][:, 0:1]
        pooled = (p0a + p1a) / jnp.maximum(cnt, 1.0)
        acc = jnp.dot(r[...], w1[...], preferred_element_type=f32)
        acc += jnp.dot(pooled, w2[...], preferred_element_type=f32)
        out[...] = acc + bb[...]

    return pl.pallas_call(
        body,
        grid=(1,),
        in_specs=[
            pl.BlockSpec((G, D), lambda i: (0, 0)),
            pl.BlockSpec((1, G, D), lambda i: (0, 0, 0)),
            pl.BlockSpec((1, G, D), lambda i: (1, 0, 0)),
            pl.BlockSpec((1, G, D), lambda i: (0, 0, 0)),
            pl.BlockSpec((1, G, D), lambda i: (1, 0, 0)),
            pl.BlockSpec((D, D), lambda i: (0, 0)),
            pl.BlockSpec((D, D), lambda i: (0, 0)),
            pl.BlockSpec((1, D), lambda i: (0, 0)),
        ],
        out_specs=pl.BlockSpec((G, D), lambda i: (0, 0)),
        out_shape=jax.ShapeDtypeStruct((G, D), f32),
    )(root, pool2, pool2, cnt2, cnt2, Wr1, Wr2, blin)


def kernel(x, edge_index, batch, root_n_id,
           W1l, W1r, b1, W2l, W2r, b2, W3l, W3r, b3, Wlin, blin):
    src = edge_index[0]
    dst = edge_index[1]
    src_p = jnp.concatenate(
        [src, jnp.zeros((EPAD - E,), i32)]).reshape(32, CHUNKS, K)
    dst_p = jnp.concatenate(
        [dst, jnp.full((EPAD - E,), N, i32)]).reshape(32, CHUNKS, K)
    batch_c = jnp.concatenate(
        [batch, jnp.full((NB - N,), G, i32)]).reshape(32, 3, 128)
    batch_s = jnp.concatenate(
        [batch, jnp.full((NPAD - N,), G, i32)]).reshape(32, 4, 80)
    root_p = root_n_id.reshape(2, 128)
    zeros128 = jnp.zeros((640, D), f32)
    ones128 = jnp.ones((128, D), f32)
    x_p = jnp.pad(x, ((0, NPAD - N), (0, 0)))

    deg2, cnt2 = _deg_kernel(dst_p, batch_c, ones128, zeros128)

    srcc = src_p.reshape(EPAD // K, K)
    dstc = dst_p.reshape(EPAD // K, K)
    src0 = srcc[:16 * NC0].reshape(16, NC0, K)
    dst0 = dstc[:16 * NC0].reshape(16, NC0, K)
    src1 = srcc[16 * NC0:].reshape(16, NC1, K)
    dst1 = dstc[16 * NC0:].reshape(16, NC1, K)

    h = x_p
    for Wl, Wr, b in ((W1l, W1r, b1), (W2l, W2r, b2), (W3l, W3r, b3)):
        agg2 = _agg_kernel(h, src0, dst0, src1, dst1, zeros128)
        h = _layer_tc(agg2, deg2, h, Wl, Wr, b.reshape(1, D))

    pool2, root = _pool_kernel(h, batch_s, root_p, zeros128)
    return _final_tc(root, pool2, cnt2, Wlin[:D], Wlin[D:], blin.reshape(1, D))


# speedup vs baseline: 1.1422x; 1.1160x over previous
import functools

import jax
import jax.numpy as jnp
from jax import lax
from jax.experimental import pallas as pl
from jax.experimental.pallas import tpu as pltpu
from jax.experimental.pallas import tpu_sc as plsc

N = 10000
E = 320000
G = 256
D = 128

NPAD = 10240
K = 128
CHUNKS = 80
EPAD = 32 * CHUNKS * K
NB = 12288
GPAD = 384

f32 = jnp.float32
i32 = jnp.int32

_MESH = plsc.VectorSubcoreMesh(core_axis_name="c", subcore_axis_name="s")


@functools.partial(
    pl.kernel,
    out_type=(jax.ShapeDtypeStruct((2, NPAD, D), f32),
              jax.ShapeDtypeStruct((2, GPAD, D), f32)),
    mesh=_MESH,
    scratch_types=[
        pltpu.VMEM((CHUNKS, K), i32),
        pltpu.VMEM((3, 128), i32),
        pltpu.VMEM((128, D), f32),
        pltpu.VMEM_SHARED((NPAD, D), f32),
        pltpu.VMEM_SHARED((GPAD, D), f32),
    ],
)
def _deg_kernel(dst_hbm, batch_hbm, ones_hbm, zeros_hbm, deg_out, cnt_out,
                dstv, batchv, onesv, degsh, cntsh):
    c = lax.axis_index("c")
    s = lax.axis_index("s")
    wid = s * 2 + c
    pltpu.sync_copy(dst_hbm.at[wid], dstv)
    pltpu.sync_copy(batch_hbm.at[wid], batchv)
    pltpu.sync_copy(ones_hbm, onesv)
    pltpu.sync_copy(zeros_hbm, degsh.at[pl.ds(s * 640, 640)])
    pltpu.sync_copy(zeros_hbm.at[pl.ds(0, 24)], cntsh.at[pl.ds(s * 24, 24)])
    plsc.subcore_barrier()

    @pl.loop(0, CHUNKS)
    def _(j):
        pltpu.sync_copy(onesv, degsh.at[dstv.at[j]], add=True)

    @pl.loop(0, 3)
    def _(j):
        pltpu.sync_copy(onesv, cntsh.at[batchv.at[j]], add=True)

    plsc.subcore_barrier()
    pltpu.sync_copy(degsh.at[pl.ds(s * 640, 640)],
                    deg_out.at[c, pl.ds(s * 640, 640)])
    pltpu.sync_copy(cntsh.at[pl.ds(s * 24, 24)],
                    cnt_out.at[c, pl.ds(s * 24, 24)])


NC0 = 144
NC1 = 16


@functools.partial(
    pl.kernel,
    out_type=jax.ShapeDtypeStruct((2, NPAD, D), f32),
    mesh=_MESH,
    scratch_types=[
        pltpu.VMEM((16, K), i32),
        pltpu.VMEM((8, K), i32),
        pltpu.VMEM((K, D), f32),
        pltpu.VMEM((K, D), f32),
        pltpu.VMEM_SHARED((NPAD, D), f32),
        pltpu.SemaphoreType.DMA,
        pltpu.SemaphoreType.DMA,
        pltpu.SemaphoreType.DMA,
        pltpu.SemaphoreType.DMA,
    ],
)
def _agg_kernel(h_hbm, src0_hbm, dst0_hbm, src1_hbm, dst1_hbm, zeros_hbm,
                out_hbm, srcr, dstr, buf_a, buf_b, aggsh, sga, sgb, ssa, ssb):
    c = lax.axis_index("c")
    s = lax.axis_index("s")
    pltpu.sync_copy(zeros_hbm, aggsh.at[pl.ds(s * 640, 640)])
    plsc.subcore_barrier()

    H = K // 2

    def gather(j, buf, sem):
        r = j & 15
        pltpu.async_copy(h_hbm.at[srcr.at[r, pl.ds(0, H)]],
                         buf.at[pl.ds(0, H)], sem)
        pltpu.async_copy(h_hbm.at[srcr.at[r, pl.ds(H, H)]],
                         buf.at[pl.ds(H, H)], sem)

    def wait_gather(buf, sem):
        pltpu.make_async_copy(h_hbm.at[srcr.at[0, pl.ds(0, H)]],
                              buf.at[pl.ds(0, H)], sem).wait()
        pltpu.make_async_copy(h_hbm.at[srcr.at[0, pl.ds(0, H)]],
                              buf.at[pl.ds(0, H)], sem).wait()

    def wait_scatter(buf, sem):
        pltpu.make_async_copy(buf, aggsh.at[dstr.at[0]], sem).wait()

    def pipeline(nch, src_hbm, dst_hbm):
        ngroups = nch // 8
        pltpu.sync_copy(src_hbm.at[s, pl.ds(0, 16)], srcr)
        pltpu.sync_copy(dst_hbm.at[s, pl.ds(0, 8)], dstr)
        gather(0, buf_a, sga)

        @pl.loop(0, ngroups)
        def _(g):
            j0 = g * 8
            for b in range(0, 8, 2):
                j = j0 + b
                gather(j + 1, buf_b, sgb)
                wait_gather(buf_a, sga)
                pltpu.async_copy(buf_a, aggsh.at[dstr.at[b]], ssa, add=True)
                wait_scatter(buf_a, ssa)

                @pl.when(j + 2 < nch)
                def _():
                    gather(j + 2, buf_a, sga)

                wait_gather(buf_b, sgb)
                pltpu.async_copy(buf_b, aggsh.at[dstr.at[b + 1]], ssb,
                                 add=True)
                wait_scatter(buf_b, ssb)

            @pl.when(g < ngroups - 1)
            def _():
                pltpu.sync_copy(dst_hbm.at[s, pl.ds(j0 + 8, 8)], dstr)

            @pl.when(g < ngroups - 2)
            def _():
                half = (g & 1) * 8
                pltpu.sync_copy(src_hbm.at[s, pl.ds(j0 + 16, 8)],
                                srcr.at[pl.ds(half, 8)])

        plsc.subcore_barrier()
        pltpu.sync_copy(aggsh.at[pl.ds(s * 640, 640)],
                        out_hbm.at[c, pl.ds(s * 640, 640)])

    @pl.when(c == 0)
    def _():
        pipeline(NC0, src0_hbm, dst0_hbm)

    @pl.when(c == 1)
    def _():
        pipeline(NC1, src1_hbm, dst1_hbm)


@functools.partial(
    pl.kernel,
    out_type=(jax.ShapeDtypeStruct((2, GPAD, D), f32),
              jax.ShapeDtypeStruct((G, D), f32)),
    mesh=_MESH,
    scratch_types=[
        pltpu.VMEM((320, D), f32),
        pltpu.VMEM((4, 80), i32),
        pltpu.VMEM((128,), i32),
        pltpu.VMEM((128, D), f32),
        pltpu.VMEM_SHARED((GPAD, D), f32),
        pltpu.SemaphoreType.DMA,
    ],
)
def _pool_kernel(h_hbm, batch_hbm, root_hbm, zeros_hbm, pool_out, root_out,
                 hv, bidxv, ridxv, rootbuf, poolsh, sem):
    c = lax.axis_index("c")
    s = lax.axis_index("s")
    wid = s * 2 + c
    pltpu.sync_copy(zeros_hbm.at[pl.ds(0, 24)], poolsh.at[pl.ds(s * 24, 24)])
    pltpu.sync_copy(h_hbm.at[pl.ds(wid * 320, 320)], hv)
    pltpu.sync_copy(batch_hbm.at[wid], bidxv)
    plsc.subcore_barrier()

    @pl.loop(0, 4)
    def _(j):
        pltpu.sync_copy(hv.at[pl.ds(j * 80, 80)], poolsh.at[bidxv.at[j]],
                        add=True)

    plsc.subcore_barrier()
    pltpu.sync_copy(poolsh.at[pl.ds(s * 24, 24)],
                    pool_out.at[c, pl.ds(s * 24, 24)])

    @pl.when(wid < 2)
    def _():
        pltpu.sync_copy(root_hbm.at[wid], ridxv)
        pltpu.async_copy(h_hbm.at[ridxv], rootbuf, sem).wait()
        pltpu.sync_copy(rootbuf, root_out.at[pl.ds(wid * 128, 128)])


BLK = 512


def _layer_tc(agg2, deg2, h, Wl, Wr, b):
    def body(a0, a1, d0, d1, h_ref, wl, wr, bb, out):
        d0a = d0[...][0]
        d1a = d1[...][0]
        deg = d0a[:, 0:1] + d1a[:, 0:1]
        dinv = 1.0 / jnp.maximum(deg, 1.0)
        t = (a0[...][0] + a1[...][0]) * dinv
        acc = jnp.dot(t, wl[...], preferred_element_type=f32)
        acc += jnp.dot(h_ref[...], wr[...], preferred_element_type=f32)
        out[...] = jnp.maximum(acc + bb[...], 0.0)

    return pl.pallas_call(
        body,
        grid=(NPAD // BLK,),
        in_specs=[
            pl.BlockSpec((1, BLK, D), lambda i: (0, i, 0)),
            pl.BlockSpec((1, BLK, D), lambda i: (1, i, 0)),
            pl.BlockSpec((1, BLK, D), lambda i: (0, i, 0)),
            pl.BlockSpec((1, BLK, D), lambda i: (1, i, 0)),
            pl.BlockSpec((BLK, D), lambda i: (i, 0)),
            pl.BlockSpec((D, D), lambda i: (0, 0)),
            pl.BlockSpec((D, D), lambda i: (0, 0)),
            pl.BlockSpec((1, D), lambda i: (0, 0)),
        ],
        out_specs=pl.BlockSpec((BLK, D), lambda i: (i, 0)),
        out_shape=jax.ShapeDtypeStruct((NPAD, D), f32),
    )(agg2, agg2, deg2, deg2, h, Wl, Wr, b)


def _final_tc(root, pool2, cnt2, Wr1, Wr2, blin):
    def body(r, p0, p1, c0, c1, w1, w2, bb, out):
        p0a = p0[...][0]
        p1a = p1[...][0]
        cnt = c0[...][0][:, 0:1] + c1[...][0][:, 0:1]
        pooled = (p0a + p1a) / jnp.maximum(cnt, 1.0)
        acc = jnp.dot(r[...], w1[...], preferred_element_type=f32)
        acc += jnp.dot(pooled, w2[...], preferred_element_type=f32)
        out[...] = acc + bb[...]

    return pl.pallas_call(
        body,
        grid=(1,),
        in_specs=[
            pl.BlockSpec((G, D), lambda i: (0, 0)),
            pl.BlockSpec((1, G, D), lambda i: (0, 0, 0)),
            pl.BlockSpec((1, G, D), lambda i: (1, 0, 0)),
            pl.BlockSpec((1, G, D), lambda i: (0, 0, 0)),
            pl.BlockSpec((1, G, D), lambda i: (1, 0, 0)),
            pl.BlockSpec((D, D), lambda i: (0, 0)),
            pl.BlockSpec((D, D), lambda i: (0, 0)),
            pl.BlockSpec((1, D), lambda i: (0, 0)),
        ],
        out_specs=pl.BlockSpec((G, D), lambda i: (0, 0)),
        out_shape=jax.ShapeDtypeStruct((G, D), f32),
    )(root, pool2, pool2, cnt2, cnt2, Wr1, Wr2, blin)


def kernel(x, edge_index, batch, root_n_id,
           W1l, W1r, b1, W2l, W2r, b2, W3l, W3r, b3, Wlin, blin):
    src = edge_index[0]
    dst = edge_index[1]
    src_p = jnp.concatenate(
        [src, jnp.zeros((EPAD - E,), i32)]).reshape(32, CHUNKS, K)
    dst_p = jnp.concatenate(
        [dst, jnp.full((EPAD - E,), N, i32)]).reshape(32, CHUNKS, K)
    batch_c = jnp.concatenate(
        [batch, jnp.full((NB - N,), G, i32)]).reshape(32, 3, 128)
    batch_s = jnp.concatenate(
        [batch, jnp.full((NPAD - N,), G, i32)]).reshape(32, 4, 80)
    root_p = root_n_id.reshape(2, 128)
    zeros128 = jnp.zeros((640, D), f32)
    ones128 = jnp.ones((128, D), f32)
    x_p = jnp.pad(x, ((0, NPAD - N), (0, 0)))

    deg2, cnt2 = _deg_kernel(dst_p, batch_c, ones128, zeros128)

    srcc = src_p.reshape(EPAD // K, K)
    dstc = dst_p.reshape(EPAD // K, K)
    src0 = srcc[:16 * NC0].reshape(16, NC0, K)
    dst0 = dstc[:16 * NC0].reshape(16, NC0, K)
    src1 = srcc[16 * NC0:].reshape(16, NC1, K)
    dst1 = dstc[16 * NC0:].reshape(16, NC1, K)

    h = x_p
    for Wl, Wr, b in ((W1l, W1r, b1), (W2l, W2r, b2), (W3l, W3r, b3)):
        agg2 = _agg_kernel(h, src0, dst0, src1, dst1, zeros128)
        h = _layer_tc(agg2, deg2, h, Wl, Wr, b.reshape(1, D))

    pool2, root = _pool_kernel(h, batch_s, root_p, zeros128)
    return _final_tc(root, pool2, cnt2, Wlin[:D], Wlin[D:], blin.reshape(1, D))
